# baseline (speedup 1.0000x reference)
import functools

import jax
import jax.numpy as jnp
from jax import lax
from jax.experimental import pallas as pl
from jax.experimental.pallas import tpu as pltpu
from jax.experimental.pallas import tpu_sc as plsc

D = 512
R = 7
CH = 64
NC = 2
NS = 16
NW = NC * NS
L = 16


def _build_sc_kernel(n_total):
    b_per_w = n_total // NW
    n_ch = b_per_w // CH
    n_pairs = n_ch // 2
    mesh = plsc.VectorSubcoreMesh(core_axis_name="c", subcore_axis_name="s")

    @functools.partial(
        pl.kernel,
        mesh=mesh,
        out_type=jax.ShapeDtypeStruct((n_total, D), jnp.float32),
        scratch_types=[
            pltpu.VMEM((4, b_per_w), jnp.int32),
            pltpu.VMEM((b_per_w,), jnp.int32),
            pltpu.VMEM((2, CH, D), jnp.float32),
            pltpu.SemaphoreType.DMA,
            pltpu.SemaphoreType.DMA,
            pltpu.SemaphoreType.DMA,
            pltpu.SemaphoreType.DMA,
        ],
    )
    def k(t_hbm, x0_hbm, x1_hbm, x2_hbm, x3_hbm, out_hbm,
          xbuf, cidx, rows, gs0, gs1, ss0, ss1):
        wid = lax.axis_index("s") * NC + lax.axis_index("c")
        base = wid * b_per_w

        pltpu.sync_copy(x0_hbm.at[pl.ds(base, b_per_w)], xbuf.at[0])
        pltpu.sync_copy(x1_hbm.at[pl.ds(base, b_per_w)], xbuf.at[1])
        pltpu.sync_copy(x2_hbm.at[pl.ds(base, b_per_w)], xbuf.at[2])
        pltpu.sync_copy(x3_hbm.at[pl.ds(base, b_per_w)], xbuf.at[3])

        def cbody(i, _):
            sl = pl.ds(i * L, L)
            v = ((xbuf[0, sl] * R + xbuf[1, sl]) * R + xbuf[2, sl]) * R \
                + xbuf[3, sl]
            cidx[sl] = v
            return 0

        lax.fori_loop(0, b_per_w // L, cbody, 0)

        gsems = (gs0, gs1)
        ssems = (ss0, ss1)

        def gather(c, b):
            idx = cidx.at[pl.ds(c * CH, CH)]
            pltpu.async_copy(t_hbm.at[idx], rows.at[b], gsems[b])

        def scatter(c, b):
            pltpu.async_copy(rows.at[b], out_hbm.at[pl.ds(base + c * CH, CH)],
                             ssems[b])

        def wait_g(b):
            pltpu.make_async_copy(out_hbm.at[pl.ds(base, CH)], rows.at[b],
                                  gsems[b]).wait()

        def wait_s(b):
            pltpu.make_async_copy(rows.at[b], out_hbm.at[pl.ds(base, CH)],
                                  ssems[b]).wait()

        gather(0, 0)
        gather(1, 1)

        def pair(p, _):
            c0 = 2 * p
            wait_g(0)
            scatter(c0, 0)
            wait_g(1)
            scatter(c0 + 1, 1)

            @pl.when(p < n_pairs - 1)
            def _():
                wait_s(0)
                gather(c0 + 2, 0)
                wait_s(1)
                gather(c0 + 3, 1)

            return 0

        lax.fori_loop(0, n_pairs, pair, 0)
        wait_s(0)
        wait_s(1)

    return k


BP = 1024
K = 32


def _build_tc_kernel(n_tc):
    grid = n_tc // BP

    def body(xq_ref, w_ref, out_ref):
        xq = xq_ref[0]
        w = w_ref[...].astype(jnp.bfloat16)
        k_iota = lax.broadcasted_iota(jnp.int32, (BP, K), 1)
        c = jnp.zeros((BP, K), jnp.float32)
        for j, off in ((0, 14), (1, 14), (2, 7), (3, 0)):
            idx = xq[j].reshape(BP, 1) + off
            c = c + (idx == k_iota).astype(jnp.float32)
        out_ref[...] = jnp.dot(c.astype(jnp.bfloat16), w,
                               preferred_element_type=jnp.float32)

    return pl.pallas_call(
        body,
        grid=(grid,),
        in_specs=[
            pl.BlockSpec((1, 4, BP), lambda i: (i, 0, 0)),
            pl.BlockSpec((K, D), lambda i: (0, 0)),
        ],
        out_specs=pl.BlockSpec((BP, D), lambda i: (i, 0)),
        out_shape=jax.ShapeDtypeStruct((n_tc, D), jnp.float32),
    )


N_SC = 196608


def kernel(x, hour_w, weekday_w, day_w, month_w):
    del month_w
    b, s, _ = x.shape
    n = b * s
    x = x.astype(jnp.int32)

    xf = x.reshape(n, 5)
    n_sc = N_SC
    n_tc = n - n_sc
    parts = []

    if n_sc:
        t = (day_w[:R][:, None, None, None, :]
             + day_w[:R][None, :, None, None, :]
             + weekday_w[:R][None, None, :, None, :]
             + hour_w[:R][None, None, None, :, :]).reshape(R ** 4, D)
        sc_k = _build_sc_kernel(n_sc)
        parts.append(sc_k(t,
                          xf[:n_sc, 0].ravel(), xf[:n_sc, 1].ravel(),
                          xf[:n_sc, 2].ravel(), xf[:n_sc, 3].ravel()))

    if n_tc:
        w_stack = jnp.concatenate(
            [hour_w[:R], weekday_w[:R], day_w[:R],
             jnp.zeros((K - 3 * R, D), jnp.float32)], axis=0)
        xq = jnp.stack([xf[n_sc:, 0], xf[n_sc:, 1],
                        xf[n_sc:, 2], xf[n_sc:, 3]])
        xq = xq.reshape(4, n_tc // BP, BP).transpose(1, 0, 2)
        tc_k = _build_tc_kernel(n_tc)
        parts.append(tc_k(xq, w_stack))

    out = parts[0] if len(parts) == 1 else jnp.concatenate(parts, axis=0)
    return out.reshape(b, s, D)


# speedup vs baseline: 1.4990x; 1.4990x over previous
import functools

import jax
import jax.numpy as jnp
from jax import lax
from jax.experimental import pallas as pl
from jax.experimental.pallas import tpu as pltpu
from jax.experimental.pallas import tpu_sc as plsc

D = 512
R = 7
CH = 64
NC = 2
NS = 16
NW = NC * NS
L = 16


def _build_sc_kernel(n_total):
    b_per_w = n_total // NW
    n_ch = b_per_w // CH
    n_pairs = n_ch // 2
    mesh = plsc.VectorSubcoreMesh(core_axis_name="c", subcore_axis_name="s")

    @functools.partial(
        pl.kernel,
        mesh=mesh,
        out_type=jax.ShapeDtypeStruct((n_total, D), jnp.float32),
        scratch_types=[
            pltpu.VMEM((4, b_per_w), jnp.int32),
            pltpu.VMEM((b_per_w,), jnp.int32),
            pltpu.VMEM((2, CH, D), jnp.float32),
            pltpu.SemaphoreType.DMA,
            pltpu.SemaphoreType.DMA,
            pltpu.SemaphoreType.DMA,
            pltpu.SemaphoreType.DMA,
        ],
    )
    def k(t_hbm, x0_hbm, x1_hbm, x2_hbm, x3_hbm, out_hbm,
          xbuf, cidx, rows, gs0, gs1, ss0, ss1):
        wid = lax.axis_index("s") * NC + lax.axis_index("c")
        base = wid * b_per_w

        pltpu.sync_copy(x0_hbm.at[pl.ds(base, b_per_w)], xbuf.at[0])
        pltpu.sync_copy(x1_hbm.at[pl.ds(base, b_per_w)], xbuf.at[1])
        pltpu.sync_copy(x2_hbm.at[pl.ds(base, b_per_w)], xbuf.at[2])
        pltpu.sync_copy(x3_hbm.at[pl.ds(base, b_per_w)], xbuf.at[3])

        def cbody(i, _):
            sl = pl.ds(i * L, L)
            v = ((xbuf[0, sl] * R + xbuf[1, sl]) * R + xbuf[2, sl]) * R \
                + xbuf[3, sl]
            cidx[sl] = v
            return 0

        gsems = (gs0, gs1)
        ssems = (ss0, ss1)

        def gather(c, b):
            idx = cidx.at[pl.ds(c * CH, CH)]
            pltpu.async_copy(t_hbm.at[idx], rows.at[b], gsems[b])

        def scatter(c, b):
            pltpu.async_copy(rows.at[b], out_hbm.at[pl.ds(base + c * CH, CH)],
                             ssems[b])

        def wait_g(b):
            pltpu.make_async_copy(out_hbm.at[pl.ds(base, CH)], rows.at[b],
                                  gsems[b]).wait()

        def wait_s(b):
            pltpu.make_async_copy(rows.at[b], out_hbm.at[pl.ds(base, CH)],
                                  ssems[b]).wait()

        lax.fori_loop(0, 2 * CH // L, cbody, 0)
        gather(0, 0)
        gather(1, 1)
        lax.fori_loop(2 * CH // L, b_per_w // L, cbody, 0)

        def pair(p, _):
            c0 = 2 * p
            wait_g(0)
            scatter(c0, 0)
            wait_g(1)
            scatter(c0 + 1, 1)

            @pl.when(p < n_pairs - 1)
            def _():
                wait_s(0)
                gather(c0 + 2, 0)
                wait_s(1)
                gather(c0 + 3, 1)

            return 0

        lax.fori_loop(0, n_pairs, pair, 0)
        wait_s(0)
        wait_s(1)

    return k


def kernel(x, hour_w, weekday_w, day_w, month_w):
    del month_w
    b, s, _ = x.shape
    n = b * s
    x = x.astype(jnp.int32)

    t = (day_w[:R][:, None, None, None, :]
         + day_w[:R][None, :, None, None, :]
         + weekday_w[:R][None, None, :, None, :]
         + hour_w[:R][None, None, None, :, :]).reshape(R ** 4, D)

    xf = x.reshape(n, 5)
    sc_k = _build_sc_kernel(n)
    out = sc_k(t, xf[:, 0].ravel(), xf[:, 1].ravel(),
               xf[:, 2].ravel(), xf[:, 3].ravel())
    return out.reshape(b, s, D)


# speedup vs baseline: 1.5014x; 1.0016x over previous
import functools

import jax
import jax.numpy as jnp
from jax import lax
from jax.experimental import pallas as pl
from jax.experimental.pallas import tpu as pltpu
from jax.experimental.pallas import tpu_sc as plsc

D = 512
R = 7
CH = 48
NB = 4
NC = 2
NS = 16
NW = NC * NS
L = 16


def _build_sc_kernel(n_total):
    b_per_w = n_total // NW
    n_ch = b_per_w // CH
    n_grp = n_ch // NB
    mesh = plsc.VectorSubcoreMesh(core_axis_name="c", subcore_axis_name="s")

    @functools.partial(
        pl.kernel,
        mesh=mesh,
        out_type=jax.ShapeDtypeStruct((n_total, D), jnp.float32),
        scratch_types=[
            pltpu.VMEM((b_per_w,), jnp.int32),
            pltpu.VMEM((b_per_w,), jnp.int32),
            pltpu.VMEM((NB, CH, D), jnp.float32),
            [pltpu.SemaphoreType.DMA] * NB,
            [pltpu.SemaphoreType.DMA] * NB,
        ],
    )
    def k(t_hbm, x0_hbm, x1_hbm, x2_hbm, x3_hbm, out_hbm,
          cidx, tmp, rows, gsems, ssems):
        wid = lax.axis_index("s") * NC + lax.axis_index("c")
        base = wid * b_per_w

        pltpu.sync_copy(x0_hbm.at[pl.ds(base, b_per_w)], cidx)

        def acc_pass(x_hbm):
            pltpu.sync_copy(x_hbm.at[pl.ds(base, b_per_w)], tmp)

            def body(i, _):
                sl = pl.ds(i * L, L)
                cidx[sl] = cidx[sl] * R + tmp[sl]
                return 0

            lax.fori_loop(0, b_per_w // L, body, 0)

        acc_pass(x1_hbm)
        acc_pass(x2_hbm)
        acc_pass(x3_hbm)

        def gather(c, b):
            idx = cidx.at[pl.ds(c * CH, CH)]
            pltpu.async_copy(t_hbm.at[idx], rows.at[b], gsems[b])

        def scatter(c, b):
            pltpu.async_copy(rows.at[b], out_hbm.at[pl.ds(base + c * CH, CH)],
                             ssems[b])

        def wait_g(b):
            pltpu.make_async_copy(out_hbm.at[pl.ds(base, CH)], rows.at[b],
                                  gsems[b]).wait()

        def wait_s(b):
            pltpu.make_async_copy(rows.at[b], out_hbm.at[pl.ds(base, CH)],
                                  ssems[b]).wait()

        for b in range(NB):
            gather(b, b)

        def grp(p, _):
            c0 = NB * p
            for b in range(NB):
                wait_g(b)
                scatter(c0 + b, b)

            @pl.when(p < n_grp - 1)
            def _():
                for b in range(NB):
                    wait_s(b)
                    gather(c0 + NB + b, b)

            return 0

        lax.fori_loop(0, n_grp, grp, 0)
        for b in range(NB):
            wait_s(b)

    return k


def kernel(x, hour_w, weekday_w, day_w, month_w):
    del month_w
    b, s, _ = x.shape
    n = b * s
    x = x.astype(jnp.int32)

    t = (day_w[:R][:, None, None, None, :]
         + day_w[:R][None, :, None, None, :]
         + weekday_w[:R][None, None, :, None, :]
         + hour_w[:R][None, None, None, :, :]).reshape(R ** 4, D)

    xf = x.reshape(n, 5)
    sc_k = _build_sc_kernel(n)
    out = sc_k(t, xf[:, 0].ravel(), xf[:, 1].ravel(),
               xf[:, 2].ravel(), xf[:, 3].ravel())
    return out.reshape(b, s, D)
